# Initial kernel scaffold; baseline (speedup 1.0000x reference)
import jax
import jax.numpy as jnp
from jax.experimental import pallas as pl


def kernel(pooled, gate_w, W1, b1, W2, b2, class_label):
    raise NotImplementedError("write your pallas kernel here")


# speedup vs baseline: 2.6316x; 2.6316x over previous
import functools

import jax
import jax.numpy as jnp
from jax import lax
from jax.experimental import pallas as pl
from jax.experimental.pallas import tpu as pltpu
from jax.experimental.pallas import tpu_sc as plsc

E_ = 8
H_ = 1024
B_ = 2048
M_ = 128
NT_ = 24
BP_ = NT_ * M_
NW_ = 32
TPW_ = B_ // NW_


def _router_body(pooled_ref, gate_ref, dest_ref, te_ref, w_ref, aux_ref):
    x = pooled_ref[...]
    gw = gate_ref[...]
    logits = lax.dot_general(gw, x, (((1,), (1,)), ((), ())),
                             preferred_element_type=jnp.float32)
    m = jnp.max(logits, axis=0, keepdims=True)
    ex = jnp.exp(logits - m)
    s = jnp.sum(ex, axis=0, keepdims=True)
    prob = ex / s
    iota_e = lax.broadcasted_iota(jnp.int32, (E_, B_), 0)
    is_max = logits == m
    idxv = jnp.min(jnp.where(is_max, iota_e, E_), axis=0, keepdims=True)
    oh = (iota_e == idxv).astype(jnp.float32)

    pmax = 1.0 / s
    w_ref[...] = pmax + (1.0 - pmax)

    lse = m + jnp.log(s)
    z_sum = jnp.sum(lse * lse, axis=1, keepdims=True)
    prob_sum = jnp.sum(prob, axis=1, keepdims=True)
    counts = jnp.sum(oh, axis=1, keepdims=True)
    bal = jnp.sum(prob_sum * counts, axis=0, keepdims=True)
    aux_ref[...] = jnp.concatenate(
        [bal * (E_ / (B_ * float(B_))), z_sum / B_, jnp.zeros((1, 6), jnp.float32)],
        axis=1)

    bw = 256
    r_i = lax.broadcasted_iota(jnp.int32, (bw, bw), 0)
    c_i = lax.broadcasted_iota(jnp.int32, (bw, bw), 1)
    tri = (r_i <= c_i).astype(jnp.float32)
    carry = jnp.zeros((E_, 1), jnp.float32)
    blocks = []
    for b in range(B_ // bw):
        ohb = oh[:, b * bw:(b + 1) * bw]
        posb = lax.dot_general(ohb, tri, (((1,), (0,)), ((), ())),
                               preferred_element_type=jnp.float32) + carry
        blocks.append(posb)
        carry = carry + jnp.sum(ohb, axis=1, keepdims=True)
    posincl = jnp.concatenate(blocks, axis=1)

    counts_i = carry.astype(jnp.int32)
    rc = ((counts_i + (M_ - 1)) // M_) * M_
    lo_i = lax.broadcasted_iota(jnp.int32, (E_, E_), 0)
    lo_j = lax.broadcasted_iota(jnp.int32, (E_, E_), 1)
    ltri = (lo_j < lo_i).astype(jnp.float32)
    off = lax.dot_general(ltri, rc.astype(jnp.float32), (((1,), (0,)), ((), ())),
                          preferred_element_type=jnp.float32)

    dest = jnp.sum(oh * (off + posincl - 1.0), axis=0, keepdims=True)
    dest_ref[...] = dest.astype(jnp.int32)

    off_next = (off + rc.astype(jnp.float32)).astype(jnp.int32)
    ti = lax.broadcasted_iota(jnp.int32, (E_, NT_), 1) * M_
    te = jnp.sum((off_next <= ti).astype(jnp.int32), axis=0, keepdims=True)
    te_ref[...] = jnp.minimum(te, E_ - 1)


def _router_call(pooled, gate_w):
    return pl.pallas_call(
        _router_body,
        out_shape=[
            jax.ShapeDtypeStruct((1, B_), jnp.int32),
            jax.ShapeDtypeStruct((1, NT_), jnp.int32),
            jax.ShapeDtypeStruct((1, B_), jnp.float32),
            jax.ShapeDtypeStruct((1, 8), jnp.float32),
        ],
    )(pooled, gate_w)


@functools.cache
def _sc_mesh():
    return plsc.VectorSubcoreMesh(core_axis_name="c", subcore_axis_name="s",
                                  num_cores=2)


@functools.cache
def _scatter_rows_kernel():
    @functools.partial(
        pl.kernel,
        out_type=jax.ShapeDtypeStruct((BP_, H_), jnp.float32),
        mesh=_sc_mesh(),
        scratch_types=[
            pltpu.VMEM((TPW_,), jnp.int32),
            pltpu.VMEM((TPW_, H_), jnp.float32),
            pltpu.SemaphoreType.DMA,
        ],
    )
    def _scatter_rows(pooled_hbm, dest_hbm, xs_hbm, idx_v, rows_v, sem):
        wid = lax.axis_index("s") * 2 + lax.axis_index("c")
        base = wid * TPW_
        pltpu.sync_copy(dest_hbm.at[pl.ds(base, TPW_)], idx_v)
        pltpu.sync_copy(pooled_hbm.at[pl.ds(base, TPW_)], rows_v)
        pltpu.async_copy(rows_v, xs_hbm.at[idx_v], sem).wait()

    return _scatter_rows


def _ffn_body(te_ref, xs_ref, w1_ref, b1_ref, w2_ref, b2_ref, ys_ref):
    i = pl.program_id(0)
    e = te_ref[i]
    x = xs_ref[...]
    w1 = w1_ref[0]
    ohe = (lax.broadcasted_iota(jnp.int32, (1, E_), 1) == e).astype(jnp.float32)
    b1 = lax.dot_general(ohe, b1_ref[...], (((1,), (0,)), ((), ())),
                         preferred_element_type=jnp.float32)
    h = lax.dot_general(x, w1, (((1,), (1,)), ((), ())),
                        preferred_element_type=jnp.float32) + b1
    a = 0.5 * h * (1.0 + lax.erf(h * 0.7071067811865476))
    w2 = w2_ref[0]
    b2 = lax.dot_general(ohe, b2_ref[...], (((1,), (0,)), ((), ())),
                         preferred_element_type=jnp.float32)
    o = lax.dot_general(a, w2, (((1,), (1,)), ((), ())),
                        preferred_element_type=jnp.float32) + b2
    mx = jnp.max(o, axis=1, keepdims=True)
    eo = jnp.exp(o - mx)
    ys_ref[...] = eo / jnp.sum(eo, axis=1, keepdims=True)


def _ffn_call(te, xs, W1, b1, W2, b2):
    grid_spec = pltpu.PrefetchScalarGridSpec(
        num_scalar_prefetch=1,
        grid=(NT_,),
        in_specs=[
            pl.BlockSpec((M_, H_), lambda i, te: (i, 0)),
            pl.BlockSpec((1, H_, H_), lambda i, te: (te[i], 0, 0)),
            pl.BlockSpec((E_, H_), lambda i, te: (0, 0)),
            pl.BlockSpec((1, 2, H_), lambda i, te: (te[i], 0, 0)),
            pl.BlockSpec((E_, 2), lambda i, te: (0, 0)),
        ],
        out_specs=pl.BlockSpec((M_, 2), lambda i, te: (i, 0)),
    )
    return pl.pallas_call(
        _ffn_body,
        grid_spec=grid_spec,
        out_shape=jax.ShapeDtypeStruct((BP_, 2), jnp.float32),
    )(te, xs, W1, b1, W2, b2)


@functools.cache
def _gather_out_kernel():
    @functools.partial(
        pl.kernel,
        out_type=jax.ShapeDtypeStruct((2 * B_,), jnp.float32),
        mesh=_sc_mesh(),
        scratch_types=[
            pltpu.VMEM((TPW_,), jnp.int32),
            pltpu.VMEM((BP_ * 2,), jnp.float32),
            pltpu.VMEM((TPW_,), jnp.float32),
            pltpu.VMEM((TPW_,), jnp.float32),
        ],
        compiler_params=pltpu.CompilerParams(needs_layout_passes=False),
    )
    def _gather_out(ys_hbm, dest_hbm, out_hbm, idx_v, ys_v, o0_v, o1_v):
        wid = lax.axis_index("s") * 2 + lax.axis_index("c")
        base = wid * TPW_
        pltpu.sync_copy(dest_hbm.at[pl.ds(base, TPW_)], idx_v)
        pltpu.sync_copy(ys_hbm, ys_v)
        for j in range(TPW_ // 16):
            ii = idx_v[pl.ds(j * 16, 16)] * 2
            o0_v[pl.ds(j * 16, 16)] = plsc.load_gather(ys_v, [ii])
            o1_v[pl.ds(j * 16, 16)] = plsc.load_gather(ys_v, [ii + 1])
        pltpu.sync_copy(o0_v, out_hbm.at[pl.ds(base, TPW_)])
        pltpu.sync_copy(o1_v, out_hbm.at[pl.ds(B_ + base, TPW_)])

    return _gather_out


def _final_body(eo_ref, w_ref, lab_ref, ce_ref, pred_ref):
    eo = eo_ref[...]
    w = w_ref[...]
    wl = eo * w
    mx = jnp.max(wl, axis=0, keepdims=True)
    lse = mx + jnp.log(jnp.sum(jnp.exp(wl - mx), axis=0, keepdims=True))
    logp = wl - lse
    lab = lab_ref[...]
    sel = jnp.where(lab == 1, logp[1:2, :], logp[0:1, :])
    ce_ref[...] = -jnp.sum(sel, axis=1, keepdims=True) / B_
    pred_ref[...] = (wl[1:2, :] > wl[0:1, :]).astype(jnp.int32)


def _final_call(eo, w, lab):
    return pl.pallas_call(
        _final_body,
        out_shape=[
            jax.ShapeDtypeStruct((1, 1), jnp.float32),
            jax.ShapeDtypeStruct((1, B_), jnp.int32),
        ],
    )(eo, w, lab)


def kernel(pooled, gate_w, W1, b1, W2, b2, class_label):
    dest2, te2, w2d, aux = _router_call(pooled, gate_w)
    dest = dest2.reshape(B_)
    xs = _scatter_rows_kernel()(pooled, dest)
    ys = _ffn_call(te2.reshape(NT_), xs, W1, b1, W2, b2)
    eo = _gather_out_kernel()(ys.reshape(BP_ * 2), dest).reshape(2, B_)
    ce2, pred2 = _final_call(eo, w2d, class_label.reshape(1, B_).astype(jnp.int32))
    ce_loss = ce2.reshape(())
    balancing_loss = aux[0, 0]
    router_z_loss = aux[0, 1]
    loss = ce_loss + 0.01 * balancing_loss + 0.001 * router_z_loss
    pred = pred2.reshape(B_)
    return (loss, ce_loss, balancing_loss, router_z_loss, pred)
